# Initial kernel scaffold; baseline (speedup 1.0000x reference)
import jax
import jax.numpy as jnp
from jax.experimental import pallas as pl


def kernel(x, x_lstm, encoded_z_gnss, edge_index, edge_attr, W1, b1, W2, b2):
    raise NotImplementedError("write your pallas kernel here")


# speedup vs baseline: 5.1665x; 5.1665x over previous
import functools

import jax
import jax.numpy as jnp
from jax import lax
from jax.experimental import pallas as pl
from jax.experimental.pallas import tpu as pltpu
from jax.experimental.pallas import tpu_sc as plsc

N_NODES = 10000
N_EDGES = 320000
D_FEAT = 128
D_EDGE = 16
D_HID = 128
D_OUT = 128

NC = 2
NS = 16
NW = NC * NS
EDGES_PER_W = N_EDGES // NW
CHUNK = 2000
NCHUNK = EDGES_PER_W // CHUNK
N_NODES_PAD = 10240
ROWS_PER_TILE = N_NODES_PAD // NS


def _seg_sum_sc(row_idx, edge_attr):
    mesh = plsc.VectorSubcoreMesh(core_axis_name="c", subcore_axis_name="s")

    @functools.partial(
        pl.kernel,
        mesh=mesh,
        out_type=jax.ShapeDtypeStruct((NC, N_NODES_PAD, D_EDGE), jnp.float32),
        scratch_types=[
            pltpu.VMEM((CHUNK,), jnp.int32),
            pltpu.VMEM((CHUNK, D_EDGE), jnp.float32),
            pltpu.VMEM_SHARED((N_NODES_PAD, D_EDGE), jnp.float32),
        ],
        compiler_params=pltpu.CompilerParams(use_tc_tiling_on_sc=False),
    )
    def k(idx_hbm, ea_hbm, out_hbm, idx_v, ea_v, agg_sh):
        cid = lax.axis_index("c")
        sid = lax.axis_index("s")
        wid = cid * NS + sid

        zrow = jnp.zeros((D_EDGE,), jnp.float32)

        def zb(i, carry):
            ea_v[i, :] = zrow
            return carry

        lax.fori_loop(0, ROWS_PER_TILE, zb, 0)
        pltpu.sync_copy(
            ea_v.at[pl.ds(0, ROWS_PER_TILE)],
            agg_sh.at[pl.ds(sid * ROWS_PER_TILE, ROWS_PER_TILE)],
        )
        plsc.subcore_barrier()

        base = wid * EDGES_PER_W
        for j in range(NCHUNK):
            off = base + j * CHUNK
            pltpu.sync_copy(idx_hbm.at[pl.ds(off, CHUNK)], idx_v)
            pltpu.sync_copy(ea_hbm.at[pl.ds(off, CHUNK)], ea_v)
            pltpu.sync_copy(ea_v, agg_sh.at[idx_v], add=True)

        plsc.subcore_barrier()
        pltpu.sync_copy(
            agg_sh.at[pl.ds(sid * ROWS_PER_TILE, ROWS_PER_TILE)],
            out_hbm.at[cid, pl.ds(sid * ROWS_PER_TILE, ROWS_PER_TILE)],
        )

    return k(row_idx, edge_attr)


ROW_BLK = 1000
N_BLK = N_NODES // ROW_BLK


def _mlp_body(x_r, xl_r, z_r, a0_r, a1_r, wx_r, wl_r, wz_r, wa_r, b1_r,
              w2_r, b2_r, o_r):
    acc = jnp.dot(x_r[...], wx_r[...], preferred_element_type=jnp.float32)
    acc = acc + jnp.dot(xl_r[...], wl_r[...], preferred_element_type=jnp.float32)
    acc = acc + jnp.dot(z_r[...], wz_r[...], preferred_element_type=jnp.float32)
    agg = a0_r[...] + a1_r[...]
    acc = acc + jnp.dot(agg, wa_r[...], preferred_element_type=jnp.float32)
    h = jnp.maximum(acc + b1_r[...], 0.0)
    o_r[...] = jnp.dot(h, w2_r[...], preferred_element_type=jnp.float32) + b2_r[...]


def _mlp_tc(x, xl, z, a0, a1, wx, wl, wz, wa, b1, w2, b2):
    row_spec = pl.BlockSpec((ROW_BLK, D_FEAT), lambda i: (i, 0))
    agg_spec = pl.BlockSpec((ROW_BLK, D_EDGE), lambda i: (i, 0))

    def full(shape):
        return pl.BlockSpec(shape, lambda i: (0, 0))

    return pl.pallas_call(
        _mlp_body,
        grid=(N_BLK,),
        in_specs=[
            row_spec, row_spec, row_spec, agg_spec, agg_spec,
            full((D_FEAT, D_HID)), full((D_FEAT, D_HID)), full((D_FEAT, D_HID)),
            full((D_EDGE, D_HID)), full((1, D_HID)),
            full((D_HID, D_OUT)), full((1, D_OUT)),
        ],
        out_specs=pl.BlockSpec((ROW_BLK, D_OUT), lambda i: (i, 0)),
        out_shape=jax.ShapeDtypeStruct((N_NODES, D_OUT), jnp.float32),
        compiler_params=pltpu.CompilerParams(
            dimension_semantics=("arbitrary",),
        ),
    )(x, xl, z, a0, a1, wx, wl, wz, wa, b1, w2, b2)


def kernel(x, x_lstm, encoded_z_gnss, edge_index, edge_attr, W1, b1, W2, b2):
    row = edge_index[0].astype(jnp.int32)
    parts = _seg_sum_sc(row, edge_attr)[:, :N_NODES, :]
    wx = W1[0:D_FEAT]
    wl = W1[D_FEAT:2 * D_FEAT]
    wz = W1[2 * D_FEAT:3 * D_FEAT]
    wa = W1[3 * D_FEAT:]
    return _mlp_tc(
        x, x_lstm, encoded_z_gnss, parts[0], parts[1],
        wx, wl, wz, wa, b1.reshape(1, D_HID), W2, b2.reshape(1, D_OUT),
    )


# speedup vs baseline: 5.6131x; 1.0865x over previous
import functools

import jax
import jax.numpy as jnp
from jax import lax
from jax.experimental import pallas as pl
from jax.experimental.pallas import tpu as pltpu
from jax.experimental.pallas import tpu_sc as plsc

N_NODES = 10000
N_EDGES = 320000
D_FEAT = 128
D_EDGE = 16
D_HID = 128
D_OUT = 128

NC = 2
NS = 16
NW = NC * NS
EDGES_PER_W = N_EDGES // NW
CHUNK = 2000
NCHUNK = EDGES_PER_W // CHUNK
N_NODES_PAD = 10240
ROWS_PER_TILE = N_NODES_PAD // NS


def _seg_sum_sc(edge_index, edge_attr):
    mesh = plsc.VectorSubcoreMesh(core_axis_name="c", subcore_axis_name="s")

    @functools.partial(
        pl.kernel,
        mesh=mesh,
        out_type=jax.ShapeDtypeStruct((NC, N_NODES_PAD, D_EDGE), jnp.float32),
        scratch_types=[
            pltpu.VMEM((2, CHUNK), jnp.int32),
            pltpu.VMEM((2, CHUNK, D_EDGE), jnp.float32),
            pltpu.VMEM_SHARED((N_NODES_PAD, D_EDGE), jnp.float32),
            pltpu.SemaphoreType.DMA,
            pltpu.SemaphoreType.DMA,
            pltpu.SemaphoreType.DMA,
            pltpu.SemaphoreType.DMA,
        ],
        compiler_params=pltpu.CompilerParams(use_tc_tiling_on_sc=False),
    )
    def k(idx_hbm, ea_hbm, out_hbm, idx_v, ea_v, agg_sh, si0, si1, se0, se1):
        cid = lax.axis_index("c")
        sid = lax.axis_index("s")
        wid = cid * NS + sid
        base = wid * EDGES_PER_W
        sis = (si0, si1)
        ses = (se0, se1)

        def start(j):
            b = j % 2
            off = base + j * CHUNK
            ci = pltpu.async_copy(
                idx_hbm.at[0, pl.ds(off, CHUNK)], idx_v.at[b], sis[b])
            ce = pltpu.async_copy(
                ea_hbm.at[pl.ds(off, CHUNK)], ea_v.at[b], ses[b])
            return ci, ce

        pend = start(0)

        zrow = jnp.zeros((D_EDGE,), jnp.float32)

        def zb(i, carry):
            ea_v[1, i, :] = zrow
            return carry

        lax.fori_loop(0, ROWS_PER_TILE, zb, 0)
        pltpu.sync_copy(
            ea_v.at[1, pl.ds(0, ROWS_PER_TILE)],
            agg_sh.at[pl.ds(sid * ROWS_PER_TILE, ROWS_PER_TILE)],
        )
        plsc.subcore_barrier()

        for j in range(NCHUNK):
            b = j % 2
            ci, ce = pend
            ci.wait()
            ce.wait()
            if j + 1 < NCHUNK:
                pend = start(j + 1)
            pltpu.sync_copy(ea_v.at[b], agg_sh.at[idx_v.at[b]], add=True)

        plsc.subcore_barrier()
        pltpu.sync_copy(
            agg_sh.at[pl.ds(sid * ROWS_PER_TILE, ROWS_PER_TILE)],
            out_hbm.at[cid, pl.ds(sid * ROWS_PER_TILE, ROWS_PER_TILE)],
        )

    return k(edge_index, edge_attr)


ROW_BLK = 1000
N_BLK = N_NODES // ROW_BLK


def _mlp_body(x_r, xl_r, z_r, parts_r, wx_r, wl_r, wz_r, wa_r, b1_r,
              w2_r, b2_r, o_r):
    acc = jnp.dot(x_r[...], wx_r[...], preferred_element_type=jnp.float32)
    acc = acc + jnp.dot(xl_r[...], wl_r[...], preferred_element_type=jnp.float32)
    acc = acc + jnp.dot(z_r[...], wz_r[...], preferred_element_type=jnp.float32)
    agg = parts_r[0] + parts_r[1]
    acc = acc + jnp.dot(agg, wa_r[...], preferred_element_type=jnp.float32)
    h = jnp.maximum(acc + b1_r[...], 0.0)
    o_r[...] = jnp.dot(h, w2_r[...], preferred_element_type=jnp.float32) + b2_r[...]


def _mlp_tc(x, xl, z, parts, wx, wl, wz, wa, b1, w2, b2):
    row_spec = pl.BlockSpec((ROW_BLK, D_FEAT), lambda i: (i, 0))
    parts_spec = pl.BlockSpec((NC, ROW_BLK, D_EDGE), lambda i: (0, i, 0))

    def full(shape):
        return pl.BlockSpec(shape, lambda i: (0,) * len(shape))

    return pl.pallas_call(
        _mlp_body,
        grid=(N_BLK,),
        in_specs=[
            row_spec, row_spec, row_spec, parts_spec,
            full((D_FEAT, D_HID)), full((D_FEAT, D_HID)), full((D_FEAT, D_HID)),
            full((D_EDGE, D_HID)), full((1, D_HID)),
            full((D_HID, D_OUT)), full((1, D_OUT)),
        ],
        out_specs=pl.BlockSpec((ROW_BLK, D_OUT), lambda i: (i, 0)),
        out_shape=jax.ShapeDtypeStruct((N_NODES, D_OUT), jnp.float32),
        compiler_params=pltpu.CompilerParams(
            dimension_semantics=("arbitrary",),
        ),
    )(x, xl, z, parts, wx, wl, wz, wa, b1, w2, b2)


def kernel(x, x_lstm, encoded_z_gnss, edge_index, edge_attr, W1, b1, W2, b2):
    parts = _seg_sum_sc(edge_index.astype(jnp.int32), edge_attr)
    wx = W1[0:D_FEAT]
    wl = W1[D_FEAT:2 * D_FEAT]
    wz = W1[2 * D_FEAT:3 * D_FEAT]
    wa = W1[3 * D_FEAT:]
    return _mlp_tc(
        x, x_lstm, encoded_z_gnss, parts,
        wx, wl, wz, wa, b1.reshape(1, D_HID), W2, b2.reshape(1, D_OUT),
    )


# speedup vs baseline: 9.2147x; 1.6416x over previous
import functools

import jax
import jax.numpy as jnp
from jax import lax
from jax.experimental import pallas as pl
from jax.experimental.pallas import tpu as pltpu
from jax.experimental.pallas import tpu_sc as plsc

N_NODES = 10000
N_EDGES = 320000
D_FEAT = 128
D_EDGE = 16
D_HID = 128
D_OUT = 128

NC = 2
NS = 16
NGRP = 4
NFS = 4
FPT = D_EDGE // NFS
EDGES_PER_SC = N_EDGES // NC
EDGES_PER_GRP = EDGES_PER_SC // NGRP
CHUNK = 4000
NCHUNK = EDGES_PER_GRP // CHUNK
VSTEPS = CHUNK // 16
N_PAD = 10240


def _seg_sum_sc(edge_index, edge_attr_t):
    mesh = plsc.VectorSubcoreMesh(core_axis_name="c", subcore_axis_name="s")

    @functools.partial(
        pl.kernel,
        mesh=mesh,
        out_type=jax.ShapeDtypeStruct((NC, NGRP, D_EDGE, N_PAD), jnp.float32),
        scratch_types=[
            pltpu.VMEM((2, CHUNK), jnp.int32),
            pltpu.VMEM((2, FPT, CHUNK), jnp.float32),
            pltpu.VMEM((FPT, N_PAD), jnp.float32),
            pltpu.SemaphoreType.DMA,
            pltpu.SemaphoreType.DMA,
            pltpu.SemaphoreType.DMA,
            pltpu.SemaphoreType.DMA,
        ],
        compiler_params=pltpu.CompilerParams(
            use_tc_tiling_on_sc=False, needs_layout_passes=False),
    )
    def k(idx_hbm, eat_hbm, out_hbm, idx_v, val_v, agg_v,
          si0, si1, sv0, sv1):
        cid = lax.axis_index("c")
        sid = lax.axis_index("s")
        grp = sid % NGRP
        fq = sid // NGRP
        base = cid * EDGES_PER_SC + grp * EDGES_PER_GRP
        sis = (si0, si1)
        svs = (sv0, sv1)

        def start(j):
            b = j % 2
            off = base + j * CHUNK
            ci = pltpu.async_copy(
                idx_hbm.at[0, pl.ds(off, CHUNK)], idx_v.at[b], sis[b])
            cv = pltpu.async_copy(
                eat_hbm.at[pl.ds(fq * FPT, FPT), pl.ds(off, CHUNK)],
                val_v.at[b], svs[b])
            return ci, cv

        pend = start(0)

        zrow = jnp.zeros((16,), jnp.float32)

        def zb(i, carry):
            for f in range(FPT):
                agg_v[f, pl.ds(i * 16, 16)] = zrow
            return carry

        lax.fori_loop(0, N_PAD // 16, zb, 0)

        for j in range(NCHUNK):
            b = j % 2
            ci, cv = pend
            ci.wait()
            cv.wait()
            if j + 1 < NCHUNK:
                pend = start(j + 1)

            def step(i, carry):
                idxv = idx_v[b, pl.ds(i * 16, 16)]
                for f in range(FPT):
                    valv = val_v[b, f, pl.ds(i * 16, 16)]
                    plsc.addupdate_scatter(agg_v.at[f], [idxv], valv)
                return carry

            lax.fori_loop(0, VSTEPS, step, 0)

        pltpu.sync_copy(
            agg_v, out_hbm.at[cid, grp, pl.ds(fq * FPT, FPT)])

    return k(edge_index, edge_attr_t)


ROW_BLK = 1024
N_BLK = (N_NODES + ROW_BLK - 1) // ROW_BLK


def _mlp_body(x_r, xl_r, z_r, parts_r, wx_r, wl_r, wz_r, wa_r, b1_r,
              w2_r, b2_r, o_r):
    acc = jnp.dot(x_r[...], wx_r[...], preferred_element_type=jnp.float32)
    acc = acc + jnp.dot(xl_r[...], wl_r[...], preferred_element_type=jnp.float32)
    acc = acc + jnp.dot(z_r[...], wz_r[...], preferred_element_type=jnp.float32)
    p = parts_r[...]
    agg_t = p[0] + p[1]
    for i in range(2, NC * NGRP):
        agg_t = agg_t + p[i]
    acc = acc + lax.dot_general(
        agg_t, wa_r[...], (((0,), (0,)), ((), ())),
        preferred_element_type=jnp.float32)
    h = jnp.maximum(acc + b1_r[...], 0.0)
    o_r[...] = jnp.dot(h, w2_r[...], preferred_element_type=jnp.float32) + b2_r[...]


def _mlp_tc(x, xl, z, parts, wx, wl, wz, wa, b1, w2, b2):
    row_spec = pl.BlockSpec((ROW_BLK, D_FEAT), lambda i: (i, 0))
    parts_spec = pl.BlockSpec((NC * NGRP, D_EDGE, ROW_BLK), lambda i: (0, 0, i))

    def full(shape):
        return pl.BlockSpec(shape, lambda i: (0,) * len(shape))

    return pl.pallas_call(
        _mlp_body,
        grid=(N_BLK,),
        in_specs=[
            row_spec, row_spec, row_spec, parts_spec,
            full((D_FEAT, D_HID)), full((D_FEAT, D_HID)), full((D_FEAT, D_HID)),
            full((D_EDGE, D_HID)), full((1, D_HID)),
            full((D_HID, D_OUT)), full((1, D_OUT)),
        ],
        out_specs=pl.BlockSpec((ROW_BLK, D_OUT), lambda i: (i, 0)),
        out_shape=jax.ShapeDtypeStruct((N_NODES, D_OUT), jnp.float32),
        compiler_params=pltpu.CompilerParams(
            dimension_semantics=("arbitrary",),
        ),
    )(x, xl, z, parts, wx, wl, wz, wa, b1, w2, b2)


def kernel(x, x_lstm, encoded_z_gnss, edge_index, edge_attr, W1, b1, W2, b2):
    parts = _seg_sum_sc(edge_index.astype(jnp.int32), edge_attr.T)
    parts = parts.reshape(NC * NGRP, D_EDGE, N_PAD)
    wx = W1[0:D_FEAT]
    wl = W1[D_FEAT:2 * D_FEAT]
    wz = W1[2 * D_FEAT:3 * D_FEAT]
    wa = W1[3 * D_FEAT:]
    return _mlp_tc(
        x, x_lstm, encoded_z_gnss, parts,
        wx, wl, wz, wa, b1.reshape(1, D_HID), W2, b2.reshape(1, D_OUT),
    )
